# baseline (speedup 1.0000x reference)
import functools

import jax
import jax.numpy as jnp
from jax import lax
from jax.experimental import pallas as pl
from jax.experimental.pallas import tpu as pltpu
from jax.experimental.pallas import tpu_sc as plsc

N_NODES = 10000
N_EDGES = 320000
D_FEAT = 128

NC = 2
NS = 16
NW = NC * NS

W = 128
E_PER_TILE = 10240
IDX_ROWS = E_PER_TILE // W
N_IDX_PAD = NW * E_PER_TILE
WIN_LAST = (N_EDGES - (NW - 1) * E_PER_TILE) // W

N_PAD = 10240
ROWS_PER_TILE = N_PAD // NS


def _sc_partial_sums(edges, recv2d):
    mesh = plsc.VectorSubcoreMesh(
        core_axis_name="c", subcore_axis_name="s", num_cores=NC, num_subcores=NS
    )

    @functools.partial(
        pl.kernel,
        out_type=jax.ShapeDtypeStruct((NC, N_PAD, D_FEAT), jnp.float32),
        mesh=mesh,
        scratch_types=[
            pltpu.VMEM((W, D_FEAT), jnp.float32),
            pltpu.VMEM((W, D_FEAT), jnp.float32),
            pltpu.VMEM((IDX_ROWS, W), jnp.int32),
            pltpu.VMEM_SHARED((N_PAD, D_FEAT), jnp.float32),
            pltpu.SemaphoreType.DMA,
            pltpu.SemaphoreType.DMA,
            pltpu.SemaphoreType.DMA,
            pltpu.SemaphoreType.DMA,
        ],
    )
    def k(edges_hbm, recv_hbm, out_hbm, ebuf0, ebuf1, ibuf, acc,
          sl0, sl1, ss0, ss1):
        cid = lax.axis_index("c")
        sid = lax.axis_index("s")
        wid = sid * NC + cid

        zeros16 = jnp.zeros((16,), jnp.float32)

        def zero_row(i, carry):
            for j in range(D_FEAT // 16):
                ebuf0[i, pl.ds(j * 16, 16)] = zeros16
            return carry

        lax.fori_loop(0, W, zero_row, 0)
        row0 = sid * ROWS_PER_TILE
        for m in range(ROWS_PER_TILE // W):
            pltpu.sync_copy(ebuf0, acc.at[pl.ds(row0 + m * W, W)])
        plsc.subcore_barrier()

        pltpu.sync_copy(recv_hbm.at[pl.ds(wid * IDX_ROWS, IDX_ROWS)], ibuf)

        ebase = wid * E_PER_TILE
        n_win = jnp.where(wid == NW - 1, WIN_LAST, IDX_ROWS)

        def load(k_, ebuf, sem):
            pltpu.async_copy(edges_hbm.at[pl.ds(ebase + k_ * W, W)], ebuf, sem)

        def wait_load(ebuf, sem):
            pltpu.make_async_copy(edges_hbm.at[pl.ds(0, W)], ebuf, sem).wait()

        def scatter(k_, ebuf, sem):
            pltpu.async_copy(ebuf, acc.at[ibuf.at[k_]], sem, add=True)

        def wait_scatter(ebuf, sem):
            pltpu.make_async_copy(ebuf, acc.at[ibuf.at[0]], sem).wait()

        load(0, ebuf0, sl0)
        load(1, ebuf1, sl1)

        def body(j, carry):
            k_ = 2 * j
            wait_load(ebuf0, sl0)
            scatter(k_, ebuf0, ss0)
            wait_load(ebuf1, sl1)
            scatter(k_ + 1, ebuf1, ss1)
            wait_scatter(ebuf0, ss0)
            load(k_ + 2, ebuf0, sl0)
            wait_scatter(ebuf1, ss1)
            load(k_ + 3, ebuf1, sl1)
            return carry

        lax.fori_loop(0, n_win // 2 - 1, body, 0)
        last = n_win - 2
        wait_load(ebuf0, sl0)
        scatter(last, ebuf0, ss0)
        wait_load(ebuf1, sl1)
        scatter(last + 1, ebuf1, ss1)
        wait_scatter(ebuf0, ss0)
        wait_scatter(ebuf1, ss1)

        plsc.subcore_barrier()

        pltpu.sync_copy(
            acc.at[pl.ds(row0, ROWS_PER_TILE)],
            out_hbm.at[cid, pl.ds(row0, ROWS_PER_TILE)],
        )

    return k(edges, recv2d)


def _combine_kernel(p_ref, o_ref):
    o_ref[...] = p_ref[0] + p_ref[1]


def _combine(partials):
    rows = 1000
    return pl.pallas_call(
        _combine_kernel,
        grid=(N_NODES // rows,),
        in_specs=[pl.BlockSpec((NC, rows, D_FEAT), lambda i: (0, i, 0))],
        out_specs=pl.BlockSpec((rows, D_FEAT), lambda i: (i, 0)),
        out_shape=jax.ShapeDtypeStruct((N_NODES, D_FEAT), jnp.float32),
    )(partials)


@jax.jit
def kernel(edges, senders, receivers):
    del senders
    recv = receivers.astype(jnp.int32)
    recv2d = jnp.concatenate(
        [recv, jnp.zeros((N_IDX_PAD - N_EDGES,), jnp.int32)]
    ).reshape(N_IDX_PAD // W, W)
    partials = _sc_partial_sums(edges, recv2d)
    return _combine(partials)


# speedup vs baseline: 1.3350x; 1.3350x over previous
import functools

import jax
import jax.numpy as jnp
from jax import lax
from jax.experimental import pallas as pl
from jax.experimental.pallas import tpu as pltpu
from jax.experimental.pallas import tpu_sc as plsc

N_NODES = 10000
N_EDGES = 320000
D_FEAT = 128

NC = 2
NS = 16
NW = NC * NS

W = 128
E_PER_TILE = 10240
IDX_ROWS = E_PER_TILE // W
N_IDX_PAD = NW * E_PER_TILE
WIN_LAST = (N_EDGES - (NW - 1) * E_PER_TILE) // W

N_PAD = 10240
ROWS_PER_TILE = N_PAD // NS


def _sc_partial_sums(edges, recv2d):
    mesh = plsc.VectorSubcoreMesh(
        core_axis_name="c", subcore_axis_name="s", num_cores=NC, num_subcores=NS
    )

    @functools.partial(
        pl.kernel,
        out_type=jax.ShapeDtypeStruct((NC, N_PAD, D_FEAT), jnp.float32),
        mesh=mesh,
        scratch_types=[
            pltpu.VMEM((W, D_FEAT), jnp.float32),
            pltpu.VMEM((W, D_FEAT), jnp.float32),
            pltpu.VMEM((IDX_ROWS, W), jnp.int32),
            pltpu.VMEM_SHARED((N_PAD, D_FEAT), jnp.float32),
            pltpu.SemaphoreType.DMA,
            pltpu.SemaphoreType.DMA,
        ],
    )
    def k(edges_hbm, recv_hbm, out_hbm, ebuf0, ebuf1, ibuf, acc, sl0, sl1):
        cid = lax.axis_index("c")
        sid = lax.axis_index("s")
        wid = sid * NC + cid

        zeros16 = jnp.zeros((16,), jnp.float32)

        def zero_row(i, carry):
            for j in range(D_FEAT // 16):
                ebuf0[i, pl.ds(j * 16, 16)] = zeros16
            return carry

        lax.fori_loop(0, W, zero_row, 0)
        row0 = sid * ROWS_PER_TILE
        for m in range(ROWS_PER_TILE // W):
            pltpu.sync_copy(ebuf0, acc.at[pl.ds(row0 + m * W, W)])
        plsc.subcore_barrier()

        pltpu.sync_copy(recv_hbm.at[pl.ds(wid * IDX_ROWS, IDX_ROWS)], ibuf)

        ebase = wid * E_PER_TILE
        n_win = jnp.where(wid == NW - 1, WIN_LAST, IDX_ROWS)

        def load(k_, ebuf, sem):
            pltpu.async_copy(edges_hbm.at[pl.ds(ebase + k_ * W, W)], ebuf, sem)

        def wait_load(ebuf, sem):
            pltpu.make_async_copy(edges_hbm.at[pl.ds(0, W)], ebuf, sem).wait()

        def scatter(k_, ebuf):
            pltpu.sync_copy(ebuf, acc.at[ibuf.at[k_]], add=True)

        load(0, ebuf0, sl0)
        load(1, ebuf1, sl1)

        def body(j, carry):
            k_ = 2 * j
            wait_load(ebuf0, sl0)
            scatter(k_, ebuf0)
            load(k_ + 2, ebuf0, sl0)
            wait_load(ebuf1, sl1)
            scatter(k_ + 1, ebuf1)
            load(k_ + 3, ebuf1, sl1)
            return carry

        lax.fori_loop(0, n_win // 2 - 1, body, 0)
        last = n_win - 2
        wait_load(ebuf0, sl0)
        scatter(last, ebuf0)
        wait_load(ebuf1, sl1)
        scatter(last + 1, ebuf1)

        plsc.subcore_barrier()

        pltpu.sync_copy(
            acc.at[pl.ds(row0, ROWS_PER_TILE)],
            out_hbm.at[cid, pl.ds(row0, ROWS_PER_TILE)],
        )

    return k(edges, recv2d)


def _combine_kernel(p_ref, o_ref):
    o_ref[...] = p_ref[0] + p_ref[1]


def _combine(partials):
    rows = 1000
    return pl.pallas_call(
        _combine_kernel,
        grid=(N_NODES // rows,),
        in_specs=[pl.BlockSpec((NC, rows, D_FEAT), lambda i: (0, i, 0))],
        out_specs=pl.BlockSpec((rows, D_FEAT), lambda i: (i, 0)),
        out_shape=jax.ShapeDtypeStruct((N_NODES, D_FEAT), jnp.float32),
    )(partials)


@jax.jit
def kernel(edges, senders, receivers):
    del senders
    recv = receivers.astype(jnp.int32)
    recv2d = jnp.concatenate(
        [recv, jnp.zeros((N_IDX_PAD - N_EDGES,), jnp.int32)]
    ).reshape(N_IDX_PAD // W, W)
    partials = _sc_partial_sums(edges, recv2d)
    return _combine(partials)


# speedup vs baseline: 1.3539x; 1.0141x over previous
import functools

import jax
import jax.numpy as jnp
from jax import lax
from jax.experimental import pallas as pl
from jax.experimental.pallas import tpu as pltpu
from jax.experimental.pallas import tpu_sc as plsc

N_NODES = 10000
N_EDGES = 320000
D_FEAT = 128

NC = 2
NS = 16
NW = NC * NS

W = 128
E_PER_TILE = 10240
IDX_ROWS = E_PER_TILE // W
N_IDX_PAD = NW * E_PER_TILE
WIN_LAST = (N_EDGES - (NW - 1) * E_PER_TILE) // W

N_PAD = 10240
ROWS_PER_TILE = N_PAD // NS


def _sc_partial_sums(edges, recv2d):
    mesh = plsc.VectorSubcoreMesh(
        core_axis_name="c", subcore_axis_name="s", num_cores=NC, num_subcores=NS
    )

    @functools.partial(
        pl.kernel,
        out_type=jax.ShapeDtypeStruct((NC, N_PAD, D_FEAT), jnp.float32),
        mesh=mesh,
        scratch_types=[
            pltpu.VMEM((W, D_FEAT), jnp.float32),
            pltpu.VMEM((W, D_FEAT), jnp.float32),
            pltpu.VMEM((IDX_ROWS, W), jnp.int32),
            pltpu.VMEM_SHARED((N_PAD, D_FEAT), jnp.float32),
            pltpu.SemaphoreType.DMA,
            pltpu.SemaphoreType.DMA,
        ],
    )
    def k(edges_hbm, recv_hbm, out_hbm, ebuf0, ebuf1, ibuf, acc, sl0, sl1):
        cid = lax.axis_index("c")
        sid = lax.axis_index("s")
        wid = sid * NC + cid

        zeros16 = jnp.zeros((16,), jnp.float32)

        def zero_row(i, carry):
            for j in range(D_FEAT // 16):
                ebuf0[i, pl.ds(j * 16, 16)] = zeros16
            return carry

        lax.fori_loop(0, W, zero_row, 0)
        row0 = sid * ROWS_PER_TILE
        for m in range(ROWS_PER_TILE // W):
            pltpu.sync_copy(ebuf0, acc.at[pl.ds(row0 + m * W, W)])
        plsc.subcore_barrier()

        pltpu.sync_copy(recv_hbm.at[pl.ds(wid * IDX_ROWS, IDX_ROWS)], ibuf)

        ebase = wid * E_PER_TILE
        n_win = jnp.where(wid == NW - 1, WIN_LAST, IDX_ROWS)

        def load(k_, ebuf, sem):
            pltpu.async_copy(edges_hbm.at[pl.ds(ebase + k_ * W, W)], ebuf, sem)

        def wait_load(ebuf, sem):
            pltpu.make_async_copy(edges_hbm.at[pl.ds(0, W)], ebuf, sem).wait()

        def scatter(k_, ebuf):
            pltpu.sync_copy(ebuf, acc.at[ibuf.at[k_]], add=True)

        load(0, ebuf0, sl0)
        load(1, ebuf1, sl1)

        def body(j, carry):
            k_ = 2 * j
            wait_load(ebuf0, sl0)
            scatter(k_, ebuf0)
            load(k_ + 2, ebuf0, sl0)
            wait_load(ebuf1, sl1)
            scatter(k_ + 1, ebuf1)
            load(k_ + 3, ebuf1, sl1)
            return carry

        lax.fori_loop(0, n_win // 2 - 1, body, 0)
        last = n_win - 2
        wait_load(ebuf0, sl0)
        scatter(last, ebuf0)
        wait_load(ebuf1, sl1)
        scatter(last + 1, ebuf1)

        plsc.subcore_barrier()

        pltpu.sync_copy(
            acc.at[pl.ds(row0, ROWS_PER_TILE)],
            out_hbm.at[cid, pl.ds(row0, ROWS_PER_TILE)],
        )

    return k(edges, recv2d)


def _combine_kernel(p_ref, o_ref):
    o_ref[...] = p_ref[0] + p_ref[1]


def _combine(partials):
    rows = 2000
    return pl.pallas_call(
        _combine_kernel,
        grid=(N_NODES // rows,),
        in_specs=[pl.BlockSpec((NC, rows, D_FEAT), lambda i: (0, i, 0))],
        out_specs=pl.BlockSpec((rows, D_FEAT), lambda i: (i, 0)),
        out_shape=jax.ShapeDtypeStruct((N_NODES, D_FEAT), jnp.float32),
    )(partials)


@jax.jit
def kernel(edges, senders, receivers):
    del senders
    recv = receivers.astype(jnp.int32)
    recv2d = jnp.concatenate(
        [recv, jnp.zeros((N_IDX_PAD - N_EDGES,), jnp.int32)]
    ).reshape(N_IDX_PAD // W, W)
    partials = _sc_partial_sums(edges, recv2d)
    return _combine(partials)
